# baseline (device time: 113254 ns/iter reference)
import jax
import jax.numpy as jnp
from jax import lax
from jax.experimental import pallas as pl
from jax.experimental.pallas import tpu as pltpu

N_DEV = 32
WIRE = jnp.bfloat16
F32 = jnp.float32


def _fused_post_attn(attn_partial, x0, mods, W_ff1, W_ff2):
    m, n = attn_partial.shape
    chunk = m // N_DEV
    half = m // 2
    hc = N_DEV // 2
    eps = 1e-5
    ap_w = attn_partial.astype(WIRE)

    def body(ap_ref, x0_ref, mods_ref, w1_ref, w2_ref, out_ref,
             rs1_buf, ag1_buf, rs2_src, rs2_buf, ag2_buf, x1_buf,
             rs1_send, rs1_recv, ag1_send, ag1_recv,
             rs2_send, rs2_recv, ag2_send, ag2_recv):
        my = lax.axis_index("i")
        my_lo = my * chunk

        def send(peer, src_ref, dst_slot, sem_send, sem_recv):
            @pl.when(my != peer)
            def _():
                pltpu.make_async_remote_copy(
                    src_ref=src_ref,
                    dst_ref=dst_slot,
                    send_sem=sem_send.at[peer],
                    recv_sem=sem_recv.at[my],
                    device_id=(peer,),
                    device_id_type=pl.DeviceIdType.MESH,
                ).start()

        def wait_recv(src, buf, sem_recv, sem_send):
            @pl.when(my != src)
            def _():
                pltpu.make_async_remote_copy(
                    src_ref=buf.at[src], dst_ref=buf.at[src],
                    send_sem=sem_send.at[src], recv_sem=sem_recv.at[src],
                    device_id=(my,), device_id_type=pl.DeviceIdType.MESH,
                ).wait_recv()

        def wait_sends(buf, sem_send):
            for peer in range(N_DEV):
                @pl.when(my != peer)
                def _():
                    pltpu.make_async_remote_copy(
                        src_ref=buf.at[0], dst_ref=buf.at[0],
                        send_sem=sem_send.at[peer], recv_sem=sem_send.at[peer],
                        device_id=(my,), device_id_type=pl.DeviceIdType.MESH,
                    ).wait_send()

        rs1_buf[my] = ap_ref[pl.ds(my_lo, chunk), :]
        for peer in range(N_DEV):
            send(peer, ap_ref.at[pl.ds(peer * chunk, chunk), :],
                 rs1_buf.at[my], rs1_send, rs1_recv)
        for src in range(N_DEV):
            wait_recv(src, rs1_buf, rs1_recv, rs1_send)
        acc1 = jnp.sum(rs1_buf[...].astype(F32), axis=0)

        ag1_buf[my] = acc1.astype(WIRE)
        for peer in range(N_DEV):
            send(peer, ag1_buf.at[my], ag1_buf.at[my], ag1_send, ag1_recv)

        def ffn_half(b):
            lo = b * half
            for src in range(b * hc, (b + 1) * hc):
                wait_recv(src, ag1_buf, ag1_recv, ag1_send)
            a1 = ag1_buf[b * hc:(b + 1) * hc].reshape(half, n).astype(F32)
            ga = mods_ref[b:b + 1, :]
            sm = mods_ref[2 + b:3 + b, :]
            shm = mods_ref[4 + b:5 + b, :]
            x1 = x0_ref[lo:lo + half, :] + ga * a1
            x1_buf[lo:lo + half, :] = x1
            mu = jnp.mean(x1, axis=-1, keepdims=True)
            var = jnp.mean((x1 - mu) * (x1 - mu), axis=-1, keepdims=True)
            xm = ((x1 - mu) / jnp.sqrt(var + eps)) * (1.0 + sm) + shm
            h = jnp.dot(xm, w1_ref[...], preferred_element_type=F32)
            h = h * (1.0 / (1.0 + jnp.exp(-h)))
            ffp = jnp.dot(h, w2_ref[...], preferred_element_type=F32)
            rs2_src[b * hc:(b + 1) * hc] = ffp.reshape(hc, chunk, n).astype(WIRE)
            for peer in range(b * hc, (b + 1) * hc):
                send(peer, rs2_src.at[peer], rs2_buf.at[my], rs2_send, rs2_recv)

        ffn_half(0)
        ffn_half(1)

        rs2_buf[my] = rs2_src[my]
        for src in range(N_DEV):
            wait_recv(src, rs2_buf, rs2_recv, rs2_send)
        acc2 = jnp.sum(rs2_buf[...].astype(F32), axis=0)

        ag2_buf[my] = acc2.astype(WIRE)
        for peer in range(N_DEV):
            send(peer, ag2_buf.at[my], ag2_buf.at[my], ag2_send, ag2_recv)
        for src in range(N_DEV):
            wait_recv(src, ag2_buf, ag2_recv, ag2_send)

        ff = ag2_buf[...].reshape(m, n).astype(F32)
        out_ref[0:half, :] = x1_buf[0:half, :] + mods_ref[6:7, :] * ff[0:half, :]
        out_ref[half:m, :] = x1_buf[half:m, :] + mods_ref[7:8, :] * ff[half:m, :]
        gm_my = jnp.where(my < hc, mods_ref[6:7, :], mods_ref[7:8, :])
        out_ref[pl.ds(my_lo, chunk), :] = (
            x1_buf[pl.ds(my_lo, chunk), :] + gm_my * acc2)

        wait_sends(rs1_buf, rs1_send)
        wait_sends(ag1_buf, ag1_send)
        wait_sends(rs2_buf, rs2_send)
        wait_sends(ag2_buf, ag2_send)

    return pl.pallas_call(
        body,
        out_shape=jax.ShapeDtypeStruct((m, n), F32),
        in_specs=[pl.BlockSpec(memory_space=pltpu.VMEM)] * 5,
        out_specs=pl.BlockSpec(memory_space=pltpu.VMEM),
        scratch_shapes=[
            pltpu.VMEM((N_DEV, chunk, n), WIRE),
            pltpu.VMEM((N_DEV, chunk, n), WIRE),
            pltpu.VMEM((N_DEV, chunk, n), WIRE),
            pltpu.VMEM((N_DEV, chunk, n), WIRE),
            pltpu.VMEM((N_DEV, chunk, n), WIRE),
            pltpu.VMEM((m, n), F32),
        ] + [pltpu.SemaphoreType.DMA((N_DEV,))] * 8,
    )(ap_w, x0, mods, W_ff1, W_ff2)


def kernel(x, Wq, Wk, Wv, Wo, t_emb, W_mod, W_ff1, W_ff2):
    B, S, D = x.shape
    eps = 1e-5
    Dh = 96
    Hq = Wq.shape[1] // Dh

    mod = t_emb @ W_mod
    sa, sha, ga, sm, shm, gm = jnp.split(mod, 6, axis=-1)

    x0 = x
    mu = x0.mean(axis=-1, keepdims=True)
    var = x0.var(axis=-1, keepdims=True)
    xa = ((x0 - mu) / jnp.sqrt(var + eps)) * (1.0 + sa[:, None, :]) + sha[:, None, :]

    Q = (xa @ Wq).reshape(B, S, Hq, Dh)
    K = (xa @ Wk).reshape(B, S, Hq, Dh)
    V = (xa @ Wv).reshape(B, S, Hq, Dh)
    scores = jnp.einsum("bihd,bjhd->bhij", Q, K) * 0.10206207261596577
    p = jax.nn.softmax(scores, axis=-1)
    attn = jnp.einsum("bhij,bjhd->bihd", p, V).reshape(B, S, Hq * Dh)
    attn_partial = attn @ Wo

    mods = jnp.concatenate([ga, sm, shm, gm], axis=0)
    out = _fused_post_attn(
        attn_partial.reshape(B * S, D), x0.reshape(B * S, D), mods, W_ff1, W_ff2
    )
    return out.reshape(B, S, D)


# device time: 112570 ns/iter; 1.0061x vs baseline; 1.0061x over previous
import jax
import jax.numpy as jnp
from jax import lax
from jax.experimental import pallas as pl
from jax.experimental.pallas import tpu as pltpu

N_DEV = 32
WIRE = jnp.bfloat16
F32 = jnp.float32


def _fused_post_attn(attn_partial, x0, mods, W_ff1, W_ff2):
    m, n = attn_partial.shape
    chunk = m // N_DEV
    half = m // 2
    hc = N_DEV // 2
    eps = 1e-5
    ap_w = attn_partial.astype(WIRE)

    def body(ap_ref, x0_ref, mods_ref, w1_ref, w2_ref, out_ref,
             rs1_buf, ag1_buf, rs2_src, rs2_buf, ag2_buf, x1_buf,
             rs1_send, rs1_recv, ag1_send, ag1_recv,
             rs2_send, rs2_recv, ag2_send, ag2_recv, bar_sems):
        my = lax.axis_index("i")
        my_lo = my * chunk

        with jax.named_scope("barrier"):
            gbar = pltpu.get_barrier_semaphore()
            for r in range(5):
                peer = lax.rem(my + (1 << r), N_DEV)
                sem = gbar if r == 0 else bar_sems.at[r]
                pl.semaphore_signal(
                    sem, inc=1, device_id=(peer,),
                    device_id_type=pl.DeviceIdType.MESH,
                )
                pl.semaphore_wait(sem, 1)

        def send(peer, src_ref, dst_slot, sem_send, sem_recv):
            @pl.when(my != peer)
            def _():
                pltpu.make_async_remote_copy(
                    src_ref=src_ref,
                    dst_ref=dst_slot,
                    send_sem=sem_send.at[peer],
                    recv_sem=sem_recv.at[my],
                    device_id=(peer,),
                    device_id_type=pl.DeviceIdType.MESH,
                ).start()

        def wait_recv(src, buf, sem_recv, sem_send):
            @pl.when(my != src)
            def _():
                pltpu.make_async_remote_copy(
                    src_ref=buf.at[src], dst_ref=buf.at[src],
                    send_sem=sem_send.at[src], recv_sem=sem_recv.at[src],
                    device_id=(my,), device_id_type=pl.DeviceIdType.MESH,
                ).wait_recv()

        def wait_sends(buf, sem_send):
            for peer in range(N_DEV):
                @pl.when(my != peer)
                def _():
                    pltpu.make_async_remote_copy(
                        src_ref=buf.at[0], dst_ref=buf.at[0],
                        send_sem=sem_send.at[peer], recv_sem=sem_send.at[peer],
                        device_id=(my,), device_id_type=pl.DeviceIdType.MESH,
                    ).wait_send()

        with jax.named_scope("rs1_send"):
            rs1_buf[my] = ap_ref[pl.ds(my_lo, chunk), :]
            for peer in range(N_DEV):
                send(peer, ap_ref.at[pl.ds(peer * chunk, chunk), :],
                     rs1_buf.at[my], rs1_send, rs1_recv)
        with jax.named_scope("rs1_wait"):
            for src in range(N_DEV):
                wait_recv(src, rs1_buf, rs1_recv, rs1_send)
        with jax.named_scope("rs1_sum"):
            acc1 = jnp.sum(rs1_buf[...].astype(F32), axis=0)

            ag1_buf[my] = acc1.astype(WIRE)
        with jax.named_scope("ag1_send"):
            for peer in range(N_DEV):
                send(peer, ag1_buf.at[my], ag1_buf.at[my], ag1_send, ag1_recv)

        def ffn_half(b):
            lo = b * half
            for src in range(b * hc, (b + 1) * hc):
                wait_recv(src, ag1_buf, ag1_recv, ag1_send)
            a1 = ag1_buf[b * hc:(b + 1) * hc].reshape(half, n).astype(F32)
            ga = mods_ref[b:b + 1, :]
            sm = mods_ref[2 + b:3 + b, :]
            shm = mods_ref[4 + b:5 + b, :]
            x1 = x0_ref[lo:lo + half, :] + ga * a1
            x1_buf[lo:lo + half, :] = x1
            mu = jnp.mean(x1, axis=-1, keepdims=True)
            var = jnp.mean((x1 - mu) * (x1 - mu), axis=-1, keepdims=True)
            xm = ((x1 - mu) / jnp.sqrt(var + eps)) * (1.0 + sm) + shm
            h = jnp.dot(xm, w1_ref[...], preferred_element_type=F32)
            h = h * (1.0 / (1.0 + jnp.exp(-h)))
            ffp = jnp.dot(h, w2_ref[...], preferred_element_type=F32)
            rs2_src[b * hc:(b + 1) * hc] = ffp.reshape(hc, chunk, n).astype(WIRE)
            for peer in range(b * hc, (b + 1) * hc):
                send(peer, rs2_src.at[peer], rs2_buf.at[my], rs2_send, rs2_recv)

        with jax.named_scope("ffn_half0"):
            ffn_half(0)
        with jax.named_scope("ffn_half1"):
            ffn_half(1)

        with jax.named_scope("rs2_wait"):
            rs2_buf[my] = rs2_src[my]
            for src in range(N_DEV):
                wait_recv(src, rs2_buf, rs2_recv, rs2_send)
        with jax.named_scope("rs2_sum"):
            acc2 = jnp.sum(rs2_buf[...].astype(F32), axis=0)

            ag2_buf[my] = acc2.astype(WIRE)
        with jax.named_scope("ag2_send"):
            for peer in range(N_DEV):
                send(peer, ag2_buf.at[my], ag2_buf.at[my], ag2_send, ag2_recv)
        with jax.named_scope("ag2_wait"):
            for src in range(N_DEV):
                wait_recv(src, ag2_buf, ag2_recv, ag2_send)

        with jax.named_scope("assemble"):
            ff = ag2_buf[...].reshape(m, n).astype(F32)
            out_ref[0:half, :] = x1_buf[0:half, :] + mods_ref[6:7, :] * ff[0:half, :]
            out_ref[half:m, :] = x1_buf[half:m, :] + mods_ref[7:8, :] * ff[half:m, :]
            gm_my = jnp.where(my < hc, mods_ref[6:7, :], mods_ref[7:8, :])
            out_ref[pl.ds(my_lo, chunk), :] = (
                x1_buf[pl.ds(my_lo, chunk), :] + gm_my * acc2)

        with jax.named_scope("drain"):
            wait_sends(rs1_buf, rs1_send)
            wait_sends(ag1_buf, ag1_send)
            wait_sends(rs2_buf, rs2_send)
            wait_sends(ag2_buf, ag2_send)

    return pl.pallas_call(
        body,
        out_shape=jax.ShapeDtypeStruct((m, n), F32),
        in_specs=[pl.BlockSpec(memory_space=pltpu.VMEM)] * 5,
        out_specs=pl.BlockSpec(memory_space=pltpu.VMEM),
        scratch_shapes=[
            pltpu.VMEM((N_DEV, chunk, n), WIRE),
            pltpu.VMEM((N_DEV, chunk, n), WIRE),
            pltpu.VMEM((N_DEV, chunk, n), WIRE),
            pltpu.VMEM((N_DEV, chunk, n), WIRE),
            pltpu.VMEM((N_DEV, chunk, n), WIRE),
            pltpu.VMEM((m, n), F32),
        ] + [pltpu.SemaphoreType.DMA((N_DEV,))] * 8
          + [pltpu.SemaphoreType.REGULAR((5,))],
        compiler_params=pltpu.CompilerParams(collective_id=0),
    )(ap_w, x0, mods, W_ff1, W_ff2)


def kernel(x, Wq, Wk, Wv, Wo, t_emb, W_mod, W_ff1, W_ff2):
    B, S, D = x.shape
    eps = 1e-5
    Dh = 96
    Hq = Wq.shape[1] // Dh

    mod = t_emb @ W_mod
    sa, sha, ga, sm, shm, gm = jnp.split(mod, 6, axis=-1)

    x0 = x
    mu = x0.mean(axis=-1, keepdims=True)
    var = x0.var(axis=-1, keepdims=True)
    xa = ((x0 - mu) / jnp.sqrt(var + eps)) * (1.0 + sa[:, None, :]) + sha[:, None, :]

    Q = (xa @ Wq).reshape(B, S, Hq, Dh)
    K = (xa @ Wk).reshape(B, S, Hq, Dh)
    V = (xa @ Wv).reshape(B, S, Hq, Dh)
    scores = jnp.einsum("bihd,bjhd->bhij", Q, K) * 0.10206207261596577
    p = jax.nn.softmax(scores, axis=-1)
    attn = jnp.einsum("bhij,bjhd->bihd", p, V).reshape(B, S, Hq * Dh)
    attn_partial = attn @ Wo

    mods = jnp.concatenate([ga, sm, shm, gm], axis=0)
    out = _fused_post_attn(
        attn_partial.reshape(B * S, D), x0.reshape(B * S, D), mods, W_ff1, W_ff2
    )
    return out.reshape(B, S, D)
